# speedup vs baseline: 1.1115x; 1.1115x over previous
import functools

import jax
import jax.numpy as jnp
from jax import lax
from jax.experimental import pallas as pl
from jax.experimental.pallas import tpu as pltpu
from jax.experimental.pallas import tpu_sc as plsc

FEAT = 128
N_ATOMS = 100000
NC = 2
NS = 16
NW = NC * NS
BPW = N_ATOMS // NW
CHUNK = 125
NCHUNK = BPW // CHUNK

_mesh = plsc.VectorSubcoreMesh(core_axis_name="c", subcore_axis_name="s")


@functools.partial(
    pl.kernel,
    mesh=_mesh,
    out_type=jax.ShapeDtypeStruct((NW, NCHUNK, CHUNK, FEAT), jnp.float32),
    scratch_types=[
        pltpu.VMEM((NCHUNK, CHUNK), jnp.int32),
        pltpu.VMEM((CHUNK, FEAT), jnp.float32),
        pltpu.SemaphoreType.DMA,
    ],
)
def _sc_gather(z_hbm, w_hbm, out_hbm, idx_v, rows_v, sem):
    wid = lax.axis_index("s") * NC + lax.axis_index("c")
    pltpu.sync_copy(z_hbm.at[wid], idx_v)

    def body(j, carry):
        pltpu.async_copy(w_hbm.at[idx_v.at[j]], rows_v, sem).wait()
        pltpu.sync_copy(rows_v, out_hbm.at[wid, j])
        return carry

    lax.fori_loop(0, NCHUNK, body, 0)


def kernel(z_number, atom_embed_weight):
    w_pad = atom_embed_weight.at[0].set(0.0)
    z3 = z_number.astype(jnp.int32).reshape(NW, NCHUNK, CHUNK)
    out = _sc_gather(z3, w_pad)
    s_i = out.reshape(N_ATOMS, FEAT)
    v_i = jnp.zeros((N_ATOMS, FEAT, 3), dtype=s_i.dtype)
    return (s_i, v_i)


# speedup vs baseline: 1.1124x; 1.0009x over previous
import functools

import jax
import jax.numpy as jnp
from jax import lax
from jax.experimental import pallas as pl
from jax.experimental.pallas import tpu as pltpu
from jax.experimental.pallas import tpu_sc as plsc

FEAT = 128
N_ATOMS = 100000
NC = 2
NS = 16
NW = NC * NS
BPW = N_ATOMS // NW
CHUNK = 125
NCHUNK = BPW // CHUNK

_mesh = plsc.VectorSubcoreMesh(core_axis_name="c", subcore_axis_name="s")


@functools.partial(
    pl.kernel,
    mesh=_mesh,
    out_type=jax.ShapeDtypeStruct((NW, NCHUNK, CHUNK, FEAT), jnp.float32),
    scratch_types=[
        pltpu.VMEM((NCHUNK, CHUNK), jnp.int32),
        pltpu.VMEM((2, CHUNK, FEAT), jnp.float32),
        pltpu.SemaphoreType.DMA,
        pltpu.SemaphoreType.DMA,
        pltpu.SemaphoreType.DMA,
        pltpu.SemaphoreType.DMA,
    ],
)
def _sc_gather(z_hbm, w_hbm, out_hbm, idx_v, rows_v, si0, si1, so0, so1):
    wid = lax.axis_index("s") * NC + lax.axis_index("c")
    pltpu.sync_copy(z_hbm.at[wid], idx_v)

    sem_in = (si0, si1)
    sem_out = (so0, so1)
    in_cp, out_cp = {}, {}
    in_cp[0] = pltpu.async_copy(w_hbm.at[idx_v.at[0]], rows_v.at[0], sem_in[0])
    for j in range(NCHUNK):
        b = j & 1
        nb = 1 - b
        if j + 1 < NCHUNK:
            if j - 1 in out_cp:
                out_cp[j - 1].wait()
            in_cp[j + 1] = pltpu.async_copy(
                w_hbm.at[idx_v.at[j + 1]], rows_v.at[nb], sem_in[nb])
        in_cp[j].wait()
        out_cp[j] = pltpu.async_copy(rows_v.at[b], out_hbm.at[wid, j], sem_out[b])
    out_cp[NCHUNK - 2].wait()
    out_cp[NCHUNK - 1].wait()


def kernel(z_number, atom_embed_weight):
    w_pad = atom_embed_weight.at[0].set(0.0)
    z3 = z_number.astype(jnp.int32).reshape(NW, NCHUNK, CHUNK)
    out = _sc_gather(z3, w_pad)
    s_i = out.reshape(N_ATOMS, FEAT)
    v_i = jnp.zeros((N_ATOMS, FEAT, 3), dtype=s_i.dtype)
    return (s_i, v_i)


# speedup vs baseline: 1.3225x; 1.1888x over previous
import functools

import jax
import jax.numpy as jnp
from jax import lax
from jax.experimental import pallas as pl
from jax.experimental.pallas import tpu as pltpu
from jax.experimental.pallas import tpu_sc as plsc

FEAT = 128
N_ATOMS = 100000
NC = 2
NS = 16
NW = NC * NS
BPW = N_ATOMS // NW
CHUNK = 125
NCHUNK = BPW // CHUNK

_mesh = plsc.VectorSubcoreMesh(core_axis_name="c", subcore_axis_name="s")


@functools.partial(
    pl.kernel,
    mesh=_mesh,
    compiler_params=pltpu.CompilerParams(use_tc_tiling_on_sc=False),
    out_type=jax.ShapeDtypeStruct((N_ATOMS, FEAT), jnp.float32),
    scratch_types=[
        pltpu.VMEM((NCHUNK, CHUNK), jnp.int32),
        pltpu.VMEM((2, CHUNK, FEAT), jnp.float32),
        pltpu.SemaphoreType.DMA,
        pltpu.SemaphoreType.DMA,
        pltpu.SemaphoreType.DMA,
        pltpu.SemaphoreType.DMA,
    ],
)
def _sc_gather(z_hbm, w_hbm, out_hbm, idx_v, rows_v, si0, si1, so0, so1):
    wid = lax.axis_index("s") * NC + lax.axis_index("c")
    pltpu.sync_copy(z_hbm.at[wid], idx_v)

    sem_in = (si0, si1)
    sem_out = (so0, so1)
    in_cp, out_cp = {}, {}
    in_cp[0] = pltpu.async_copy(w_hbm.at[idx_v.at[0]], rows_v.at[0], sem_in[0])
    for j in range(NCHUNK):
        b = j & 1
        nb = 1 - b
        if j + 1 < NCHUNK:
            if j - 1 in out_cp:
                out_cp[j - 1].wait()
            in_cp[j + 1] = pltpu.async_copy(
                w_hbm.at[idx_v.at[j + 1]], rows_v.at[nb], sem_in[nb])
        in_cp[j].wait()
        out_cp[j] = pltpu.async_copy(
            rows_v.at[b], out_hbm.at[pl.ds(wid * BPW + j * CHUNK, CHUNK)], sem_out[b])
    out_cp[NCHUNK - 2].wait()
    out_cp[NCHUNK - 1].wait()


def kernel(z_number, atom_embed_weight):
    w_pad = atom_embed_weight.at[0].set(0.0)
    z3 = z_number.astype(jnp.int32).reshape(NW, NCHUNK, CHUNK)
    s_i = _sc_gather(z3, w_pad)
    v_i = jnp.zeros((N_ATOMS, FEAT, 3), dtype=s_i.dtype)
    return (s_i, v_i)


# speedup vs baseline: 1.3952x; 1.0550x over previous
import functools

import jax
import jax.numpy as jnp
from jax import lax
from jax.experimental import pallas as pl
from jax.experimental.pallas import tpu as pltpu
from jax.experimental.pallas import tpu_sc as plsc

FEAT = 128
N_ATOMS = 100000
VOCAB_ROWS = 100
NC = 2
NS = 16
NW = NC * NS
BPW = N_ATOMS // NW
GRP = 16
CHUNK = 128
NFULL = BPW // CHUNK
TAIL = BPW - NFULL * CHUNK
TAILGRP = (TAIL + GRP - 1) // GRP
IDXPAD = (NFULL * CHUNK + TAILGRP * GRP)

_mesh = plsc.VectorSubcoreMesh(core_axis_name="c", subcore_axis_name="s")


@functools.partial(
    pl.kernel,
    mesh=_mesh,
    compiler_params=pltpu.CompilerParams(
        use_tc_tiling_on_sc=False, needs_layout_passes=False),
    out_type=jax.ShapeDtypeStruct((N_ATOMS * FEAT,), jnp.float32),
    scratch_types=[
        pltpu.VMEM((IDXPAD,), jnp.int32),
        pltpu.VMEM((VOCAB_ROWS * FEAT,), jnp.float32),
        pltpu.VMEM((CHUNK * FEAT,), jnp.float32),
        pltpu.VMEM((CHUNK * FEAT,), jnp.float32),
        pltpu.SemaphoreType.DMA,
        pltpu.SemaphoreType.DMA,
    ],
)
def _sc_embed(z_hbm, w_hbm, out_hbm, idx_v, table_v, buf0_v, buf1_v, so0, so1):
    wid = lax.axis_index("s") * NC + lax.axis_index("c")
    pltpu.sync_copy(w_hbm, table_v)
    pltpu.sync_copy(z_hbm.at[wid], idx_v.at[pl.ds(0, BPW)])

    zero16 = jnp.zeros((GRP,), jnp.float32)
    for k in range(FEAT // GRP):
        table_v[pl.ds(k * GRP, GRP)] = zero16

    lane = lax.iota(jnp.int32, GRP)
    sems = (so0, so1)
    base = wid * BPW * FEAT

    def make_group_body(cbase, bufref):
        def grp_body(g, carry):
            gbase = cbase + g * GRP
            rows = idx_v[pl.ds(gbase, GRP)]
            rows = jnp.minimum(jnp.maximum(rows, 0), VOCAB_ROWS - 1)
            src0 = rows * FEAT
            dst0 = (g * GRP + lane) * FEAT
            for fp in range(GRP):
                perm = jnp.bitwise_and(lane + fp, GRP - 1)
                s0 = src0 + perm
                d0 = dst0 + perm
                for k in range(FEAT // GRP):
                    val = plsc.load_gather(table_v, [s0 + k * GRP])
                    plsc.store_scatter(bufref, [d0 + k * GRP], val)
            return carry

        return grp_body

    def drain_full(bufref, sem):
        pltpu.make_async_copy(
            out_hbm.at[pl.ds(0, CHUNK * FEAT)], bufref, sem).wait()

    def do_chunk(c, bufref, sem):
        @pl.when(c >= 2)
        def _():
            drain_full(bufref, sem)

        lax.fori_loop(0, CHUNK // GRP, make_group_body(c * CHUNK, bufref), 0)
        pltpu.async_copy(
            bufref,
            out_hbm.at[pl.ds(base + c * CHUNK * FEAT, CHUNK * FEAT)],
            sem)

    def chunk_body(c, carry):
        @pl.when((c & 1) == 0)
        def _():
            do_chunk(c, buf0_v, so0)

        @pl.when((c & 1) == 1)
        def _():
            do_chunk(c, buf1_v, so1)

        return carry

    lax.fori_loop(0, NFULL, chunk_body, 0)

    drain_full(buf0_v, so0)
    lax.fori_loop(0, TAILGRP, make_group_body(NFULL * CHUNK, buf0_v), 0)
    pltpu.async_copy(
        buf0_v.at[pl.ds(0, TAIL * FEAT)],
        out_hbm.at[pl.ds(base + NFULL * CHUNK * FEAT, TAIL * FEAT)],
        so0)
    pltpu.make_async_copy(
        out_hbm.at[pl.ds(0, TAIL * FEAT)],
        buf0_v.at[pl.ds(0, TAIL * FEAT)], so0).wait()
    drain_full(buf1_v, so1)


def kernel(z_number, atom_embed_weight):
    z2 = z_number.astype(jnp.int32).reshape(NW, BPW)
    w1 = atom_embed_weight.reshape(VOCAB_ROWS * FEAT)
    out = _sc_embed(z2, w1)
    s_i = out.reshape(N_ATOMS, FEAT)
    v_i = jnp.zeros((N_ATOMS, FEAT, 3), dtype=s_i.dtype)
    return (s_i, v_i)


# speedup vs baseline: 3.1461x; 2.2549x over previous
import functools

import jax
import jax.numpy as jnp
from jax import lax
from jax.experimental import pallas as pl
from jax.experimental.pallas import tpu as pltpu
from jax.experimental.pallas import tpu_sc as plsc

FEAT = 128
N_ATOMS = 100000
VOCAB_ROWS = 100
NC = 2
NS = 16
NW = NC * NS
BPW = N_ATOMS // NW
GRP = 16
CHUNK = 128
NFULL = BPW // CHUNK
TAIL = BPW - NFULL * CHUNK
TAILGRP = (TAIL + GRP - 1) // GRP
IDXPAD = (NFULL * CHUNK + TAILGRP * GRP)

_mesh = plsc.VectorSubcoreMesh(core_axis_name="c", subcore_axis_name="s")


@functools.partial(
    pl.kernel,
    mesh=_mesh,
    compiler_params=pltpu.CompilerParams(
        use_tc_tiling_on_sc=False, needs_layout_passes=False),
    out_type=jax.ShapeDtypeStruct((N_ATOMS * FEAT,), jnp.float32),
    scratch_types=[
        pltpu.VMEM((IDXPAD,), jnp.int32),
        pltpu.VMEM((VOCAB_ROWS * FEAT,), jnp.float32),
        pltpu.VMEM((CHUNK * FEAT,), jnp.float32),
        pltpu.VMEM((CHUNK * FEAT,), jnp.float32),
        pltpu.SemaphoreType.DMA,
        pltpu.SemaphoreType.DMA,
    ],
)
def _sc_embed(z_hbm, w_hbm, out_hbm, idx_v, table_v, buf0_v, buf1_v, so0, so1):
    wid = lax.axis_index("s") * NC + lax.axis_index("c")
    pltpu.sync_copy(w_hbm, table_v)
    pltpu.sync_copy(z_hbm.at[wid], idx_v.at[pl.ds(0, BPW)])

    zero16 = jnp.zeros((GRP,), jnp.float32)
    for k in range(FEAT // GRP):
        table_v[pl.ds(k * GRP, GRP)] = zero16

    lane = lax.iota(jnp.int32, GRP)
    sems = (so0, so1)
    base = wid * BPW * FEAT

    def make_group_body(cbase, bufref):
        def grp_body(g, carry):
            gbase = cbase + g * GRP
            rows = idx_v[pl.ds(gbase, GRP)]
            rows = jnp.minimum(jnp.maximum(rows, 0), VOCAB_ROWS - 1)
            src0 = rows * FEAT
            dst0 = (g * GRP + lane) * FEAT

            @plsc.parallel_loop(0, GRP, unroll=4)
            def fp_loop(fp):
                perm = jnp.bitwise_and(lane + fp, GRP - 1)
                s0 = src0 + perm
                d0 = dst0 + perm
                vals = [plsc.load_gather(table_v, [s0 + k * GRP])
                        for k in range(FEAT // GRP)]
                for k in range(FEAT // GRP):
                    plsc.store_scatter(bufref, [d0 + k * GRP], vals[k])

            return carry

        return grp_body

    def drain_full(bufref, sem):
        pltpu.make_async_copy(
            out_hbm.at[pl.ds(0, CHUNK * FEAT)], bufref, sem).wait()

    def do_chunk(c, bufref, sem):
        @pl.when(c >= 2)
        def _():
            drain_full(bufref, sem)

        lax.fori_loop(0, CHUNK // GRP, make_group_body(c * CHUNK, bufref), 0)
        pltpu.async_copy(
            bufref,
            out_hbm.at[pl.ds(base + c * CHUNK * FEAT, CHUNK * FEAT)],
            sem)

    def chunk_body(c, carry):
        @pl.when((c & 1) == 0)
        def _():
            do_chunk(c, buf0_v, so0)

        @pl.when((c & 1) == 1)
        def _():
            do_chunk(c, buf1_v, so1)

        return carry

    lax.fori_loop(0, NFULL, chunk_body, 0)

    drain_full(buf0_v, so0)
    lax.fori_loop(0, TAILGRP, make_group_body(NFULL * CHUNK, buf0_v), 0)
    pltpu.async_copy(
        buf0_v.at[pl.ds(0, TAIL * FEAT)],
        out_hbm.at[pl.ds(base + NFULL * CHUNK * FEAT, TAIL * FEAT)],
        so0)
    pltpu.make_async_copy(
        out_hbm.at[pl.ds(0, TAIL * FEAT)],
        buf0_v.at[pl.ds(0, TAIL * FEAT)], so0).wait()
    drain_full(buf1_v, so1)


def kernel(z_number, atom_embed_weight):
    z2 = z_number.astype(jnp.int32).reshape(NW, BPW)
    w1 = atom_embed_weight.reshape(VOCAB_ROWS * FEAT)
    out = _sc_embed(z2, w1)
    s_i = out.reshape(N_ATOMS, FEAT)
    v_i = jnp.zeros((N_ATOMS, FEAT, 3), dtype=s_i.dtype)
    return (s_i, v_i)
